# Initial kernel scaffold; baseline (speedup 1.0000x reference)
import jax
import jax.numpy as jnp
from jax.experimental import pallas as pl


def kernel(x, possible_moves):
    raise NotImplementedError("write your pallas kernel here")


# speedup vs baseline: 3.8511x; 3.8511x over previous
import functools

import jax
import jax.numpy as jnp
from jax import lax
from jax.experimental import pallas as pl
from jax.experimental.pallas import tpu as pltpu
from jax.experimental.pallas import tpu_sc as plsc

B, N, K = 64, 32768, 512
NC, NS, L = 2, 16, 16
NW = NC * NS
ROWS_PER_W = B // NW
NEG = -1000000000.0

_mesh = plsc.VectorSubcoreMesh(core_axis_name="c", subcore_axis_name="s")


@functools.partial(
    pl.kernel,
    mesh=_mesh,
    out_type=jax.ShapeDtypeStruct((B, N), jnp.float32),
    scratch_types=[
        pltpu.VMEM((N,), jnp.float32),
        pltpu.VMEM((N,), jnp.float32),
        pltpu.VMEM((K,), jnp.int32),
    ],
    compiler_params=pltpu.CompilerParams(needs_layout_passes=False),
)
def _filter_moves(x_hbm, mv_hbm, out_hbm, xrow, obuf, idx):
    wid = lax.axis_index("s") * NC + lax.axis_index("c")
    neg = jnp.full((L,), NEG, jnp.float32)

    def fill(i, _):
        obuf[pl.ds(i * L, L)] = neg
        return 0

    lax.fori_loop(0, N // L, fill, 0)

    for r in range(ROWS_PER_W):
        row = wid * ROWS_PER_W + r
        pltpu.sync_copy(mv_hbm.at[row], idx)
        pltpu.sync_copy(x_hbm.at[row], xrow)
        for c in range(K // L):
            ids = idx[pl.ds(c * L, L)]
            vals = plsc.load_gather(xrow, [ids])
            vals = jnp.where(vals == 0.0, jnp.float32(NEG), vals)
            plsc.store_scatter(obuf, [ids], vals)
        pltpu.sync_copy(obuf, out_hbm.at[row])
        if r + 1 < ROWS_PER_W:
            for c in range(K // L):
                ids = idx[pl.ds(c * L, L)]
                plsc.store_scatter(obuf, [ids], neg)


def kernel(x, possible_moves):
    return _filter_moves(x, possible_moves.astype(jnp.int32))


# speedup vs baseline: 5.2081x; 1.3524x over previous
import functools

import jax
import jax.numpy as jnp
from jax import lax
from jax.experimental import pallas as pl
from jax.experimental.pallas import tpu as pltpu
from jax.experimental.pallas import tpu_sc as plsc

B, N, K = 64, 32768, 512
NC, NS, L = 2, 16, 16
NW = NC * NS
RW = B // NW
NEG = -1000000000.0

_mesh = plsc.VectorSubcoreMesh(core_axis_name="c", subcore_axis_name="s")


@functools.partial(
    pl.kernel,
    mesh=_mesh,
    out_type=jax.ShapeDtypeStruct((B, N), jnp.float32),
    scratch_types=[
        pltpu.VMEM((N,), jnp.float32),
        pltpu.VMEM((RW, N), jnp.float32),
        pltpu.VMEM((RW, K), jnp.int32),
        pltpu.SemaphoreType.DMA,
        pltpu.SemaphoreType.DMA,
        pltpu.SemaphoreType.DMA,
    ],
    compiler_params=pltpu.CompilerParams(needs_layout_passes=False),
)
def _filter_moves(x_hbm, mv_hbm, out_hbm, obuf, xrows, idx, semi, sem0, sem1):
    wid = lax.axis_index("s") * NC + lax.axis_index("c")
    neg = jnp.full((L,), NEG, jnp.float32)
    sems = [sem0, sem1]

    copies = []
    for r in range(RW):
        row = wid * RW + r
        copies.append(pltpu.async_copy(mv_hbm.at[row], idx.at[r], semi))
        copies.append(pltpu.async_copy(x_hbm.at[row], xrows.at[r], sems[r]))

    def fill(i, _):
        base = i * (8 * L)
        for j in range(8):
            obuf[pl.ds(base + j * L, L)] = neg
        return 0

    lax.fori_loop(0, N // (8 * L), fill, 0)

    for cp in copies:
        cp.wait()

    for r in range(RW):
        row = wid * RW + r
        rv = jnp.full((L,), r, jnp.int32)
        for c in range(K // L):
            iv = idx[r, pl.ds(c * L, L)]
            v = plsc.load_gather(xrows, [rv, iv])
            v = jnp.where(v == 0.0, jnp.float32(NEG), v)
            plsc.store_scatter(obuf, [iv], v)
        pltpu.sync_copy(obuf, out_hbm.at[row])
        if r + 1 < RW:
            for c in range(K // L):
                iv = idx[r, pl.ds(c * L, L)]
                plsc.store_scatter(obuf, [iv], neg)


def kernel(x, possible_moves):
    return _filter_moves(x, possible_moves.astype(jnp.int32))
